# baseline (speedup 1.0000x reference)
import numpy as np
import jax
import jax.numpy as jnp
from jax.experimental import pallas as pl
from jax.experimental.pallas import tpu as pltpu

_R = 400
_CW = 1024
_HD = 32
_C = 16


def _phase1_body(x_ref, adj_ref, w1_ref, b1_ref, w2_ref,
                 g_ref, part_ref, sg_ref):
    i = pl.program_id(0)
    r = adj_ref.shape[0]

    @pl.when(i == 0)
    def _():
        sg_ref[:, :_HD] = jnp.dot(x_ref[:], w1_ref[:],
                                  preferred_element_type=jnp.float32)
        sg_ref[:, _HD:] = jnp.zeros_like(sg_ref[:, _HD:])

    p = jnp.dot(adj_ref[:], sg_ref[:], preferred_element_type=jnp.float32,
                precision=jax.lax.Precision.DEFAULT)
    h = jnp.maximum(p[:, :_HD] + b1_ref[:], 0.0)
    gi = jnp.dot(h, w2_ref[:], preferred_element_type=jnp.float32)
    g_ref[:] = gi.astype(jnp.bfloat16)
    sg_ref[pl.ds(i * r, r), _HD:] = gi
    part_ref[:] = p[:, _HD:]


def _phase2_body(n, ii_ref, jj_ref, th_ref, fs_ref, ls_ref,
                 adj_ref, part_ref, g_ref, b2_ref, out_ref):
    t = pl.program_id(0)
    cw = adj_ref.shape[-1]
    th = th_ref[t]

    g_tile = g_ref[pl.ds(jj_ref[t] * cw, cw), :]
    rows = jax.lax.broadcasted_iota(jnp.int32, (cw, 1), 0)
    g_m = jnp.where(rows >= th, g_tile, jnp.bfloat16(0))
    cols = jax.lax.broadcasted_iota(jnp.int32, (1, cw), 1)
    a16 = jnp.where(cols < n - jj_ref[t] * cw,
                    adj_ref[:], 0.0).astype(jnp.bfloat16)
    contrib = jnp.dot(a16, g_m, preferred_element_type=jnp.float32)

    @pl.when(fs_ref[t] == 1)
    def _():
        out_ref[:] = part_ref[:] + contrib

    @pl.when(fs_ref[t] == 0)
    def _():
        out_ref[:] = out_ref[:] + contrib

    @pl.when(ls_ref[t] == 1)
    def _():
        z = out_ref[:] + b2_ref[:]
        m = jnp.max(z, axis=1, keepdims=True)
        lse = jnp.log(jnp.sum(jnp.exp(z - m), axis=1, keepdims=True))
        out_ref[:] = z - m - lse


def _upper_tiles(n, r, cw):
    nb = n // r
    ncb = -(-n // cw)
    ii, jj, th, fs, ls = [], [], [], [], []
    for i in range(nb):
        covered = i * r
        start = min(covered // cw, ncb - 1)
        js = list(range(start, ncb)) if covered < n else [ncb - 1]
        for k, j in enumerate(js):
            ii.append(i)
            jj.append(j)
            th.append(int(np.clip(covered - j * cw, 0, cw)))
            fs.append(1 if k == 0 else 0)
            ls.append(1 if k == len(js) - 1 else 0)
    mk = lambda v: jnp.asarray(np.asarray(v, np.int32))
    return mk(ii), mk(jj), mk(th), mk(fs), mk(ls)


def kernel(x, adj, W1, b1, W2, b2):
    n, f = x.shape
    hd = W1.shape[1]
    c = W2.shape[1]
    r = _R
    cw = _CW
    nb = n // r

    g, part = pl.pallas_call(
        _phase1_body,
        grid=(nb,),
        in_specs=[
            pl.BlockSpec((n, f), lambda i: (0, 0)),
            pl.BlockSpec((r, n), lambda i: (i, 0)),
            pl.BlockSpec((f, hd), lambda i: (0, 0)),
            pl.BlockSpec((1, hd), lambda i: (0, 0)),
            pl.BlockSpec((hd, c), lambda i: (0, 0)),
        ],
        out_specs=[
            pl.BlockSpec((r, c), lambda i: (i, 0)),
            pl.BlockSpec((r, c), lambda i: (i, 0)),
        ],
        out_shape=[
            jax.ShapeDtypeStruct((n, c), jnp.bfloat16),
            jax.ShapeDtypeStruct((n, c), jnp.float32),
        ],
        scratch_shapes=[
            pltpu.VMEM((n, hd + c), jnp.float32),
        ],
    )(x, adj, W1, b1.reshape(1, hd), W2)

    ii, jj, th, fs, ls = _upper_tiles(n, r, cw)
    t_steps = ii.shape[0]
    ncb = -(-n // cw)
    n_pad = ncb * cw
    g_pad = jnp.pad(g, ((0, n_pad - n), (0, 0)))

    grid_spec = pltpu.PrefetchScalarGridSpec(
        num_scalar_prefetch=5,
        grid=(t_steps,),
        in_specs=[
            pl.BlockSpec((r, cw), lambda t, ii, jj, th, fs, ls: (ii[t], jj[t])),
            pl.BlockSpec((r, c), lambda t, ii, jj, th, fs, ls: (ii[t], 0)),
            pl.BlockSpec((n_pad, c), lambda t, ii, jj, th, fs, ls: (0, 0)),
            pl.BlockSpec((1, c), lambda t, ii, jj, th, fs, ls: (0, 0)),
        ],
        out_specs=pl.BlockSpec((r, c), lambda t, ii, jj, th, fs, ls: (ii[t], 0)),
    )

    out = pl.pallas_call(
        lambda *refs: _phase2_body(n, *refs),
        grid_spec=grid_spec,
        out_shape=jax.ShapeDtypeStruct((n, c), jnp.float32),
    )(ii, jj, th, fs, ls, adj, part, g_pad, b2.reshape(1, c))

    return out


# speedup vs baseline: 1.1465x; 1.1465x over previous
import numpy as np
import jax
import jax.numpy as jnp
from jax.experimental import pallas as pl
from jax.experimental.pallas import tpu as pltpu

_R = 400
_CW = 2048
_HD = 32
_C = 16


def _phase1_body(x_ref, adj_ref, w1_ref, b1_ref, w2_ref,
                 g_ref, part_ref, sg_ref):
    i = pl.program_id(0)
    r = adj_ref.shape[0]

    @pl.when(i == 0)
    def _():
        sg_ref[:, :_HD] = jnp.dot(x_ref[:], w1_ref[:],
                                  preferred_element_type=jnp.float32)
        sg_ref[:, _HD:] = jnp.zeros_like(sg_ref[:, _HD:])

    p = jnp.dot(adj_ref[:], sg_ref[:], preferred_element_type=jnp.float32,
                precision=jax.lax.Precision.DEFAULT)
    h = jnp.maximum(p[:, :_HD] + b1_ref[:], 0.0)
    gi = jnp.dot(h, w2_ref[:], preferred_element_type=jnp.float32)
    g_ref[:] = gi.astype(jnp.bfloat16)
    sg_ref[pl.ds(i * r, r), _HD:] = gi
    part_ref[:] = p[:, _HD:]


def _phase2_body(n, ii_ref, jj_ref, th_ref, fs_ref, ls_ref,
                 adj_ref, part_ref, g_ref, b2_ref, out_ref):
    t = pl.program_id(0)
    cw = adj_ref.shape[-1]
    th = th_ref[t]

    g_tile = g_ref[pl.ds(jj_ref[t] * cw, cw), :]
    rows = jax.lax.broadcasted_iota(jnp.int32, (cw, 1), 0)
    g_m = jnp.where(rows >= th, g_tile, jnp.bfloat16(0))
    cols = jax.lax.broadcasted_iota(jnp.int32, (1, cw), 1)
    a16 = jnp.where(cols < n - jj_ref[t] * cw,
                    adj_ref[:], 0.0).astype(jnp.bfloat16)
    contrib = jnp.dot(a16, g_m, preferred_element_type=jnp.float32)

    @pl.when(fs_ref[t] == 1)
    def _():
        out_ref[:] = part_ref[:] + contrib

    @pl.when(fs_ref[t] == 0)
    def _():
        out_ref[:] = out_ref[:] + contrib

    @pl.when(ls_ref[t] == 1)
    def _():
        z = out_ref[:] + b2_ref[:]
        m = jnp.max(z, axis=1, keepdims=True)
        lse = jnp.log(jnp.sum(jnp.exp(z - m), axis=1, keepdims=True))
        out_ref[:] = z - m - lse


def _upper_tiles(n, r, cw):
    nb = n // r
    ncb = -(-n // cw)
    ii, jj, th, fs, ls = [], [], [], [], []
    for i in range(nb):
        covered = i * r
        start = min(covered // cw, ncb - 1)
        js = list(range(start, ncb)) if covered < n else [ncb - 1]
        for k, j in enumerate(js):
            ii.append(i)
            jj.append(j)
            th.append(int(np.clip(covered - j * cw, 0, cw)))
            fs.append(1 if k == 0 else 0)
            ls.append(1 if k == len(js) - 1 else 0)
    mk = lambda v: jnp.asarray(np.asarray(v, np.int32))
    return mk(ii), mk(jj), mk(th), mk(fs), mk(ls)


def kernel(x, adj, W1, b1, W2, b2):
    n, f = x.shape
    hd = W1.shape[1]
    c = W2.shape[1]
    r = _R
    cw = _CW
    nb = n // r

    g, part = pl.pallas_call(
        _phase1_body,
        grid=(nb,),
        in_specs=[
            pl.BlockSpec((n, f), lambda i: (0, 0)),
            pl.BlockSpec((r, n), lambda i: (i, 0)),
            pl.BlockSpec((f, hd), lambda i: (0, 0)),
            pl.BlockSpec((1, hd), lambda i: (0, 0)),
            pl.BlockSpec((hd, c), lambda i: (0, 0)),
        ],
        out_specs=[
            pl.BlockSpec((r, c), lambda i: (i, 0)),
            pl.BlockSpec((r, c), lambda i: (i, 0)),
        ],
        out_shape=[
            jax.ShapeDtypeStruct((n, c), jnp.bfloat16),
            jax.ShapeDtypeStruct((n, c), jnp.float32),
        ],
        scratch_shapes=[
            pltpu.VMEM((n, hd + c), jnp.float32),
        ],
    )(x, adj, W1, b1.reshape(1, hd), W2)

    ii, jj, th, fs, ls = _upper_tiles(n, r, cw)
    t_steps = ii.shape[0]
    ncb = -(-n // cw)
    n_pad = ncb * cw
    g_pad = jnp.pad(g, ((0, n_pad - n), (0, 0)))

    grid_spec = pltpu.PrefetchScalarGridSpec(
        num_scalar_prefetch=5,
        grid=(t_steps,),
        in_specs=[
            pl.BlockSpec((r, cw), lambda t, ii, jj, th, fs, ls: (ii[t], jj[t])),
            pl.BlockSpec((r, c), lambda t, ii, jj, th, fs, ls: (ii[t], 0)),
            pl.BlockSpec((n_pad, c), lambda t, ii, jj, th, fs, ls: (0, 0)),
            pl.BlockSpec((1, c), lambda t, ii, jj, th, fs, ls: (0, 0)),
        ],
        out_specs=pl.BlockSpec((r, c), lambda t, ii, jj, th, fs, ls: (ii[t], 0)),
    )

    out = pl.pallas_call(
        lambda *refs: _phase2_body(n, *refs),
        grid_spec=grid_spec,
        out_shape=jax.ShapeDtypeStruct((n, c), jnp.float32),
    )(ii, jj, th, fs, ls, adj, part, g_pad, b2.reshape(1, c))

    return out
